# baseline (speedup 1.0000x reference)
import jax
import jax.numpy as jnp
from jax.experimental import pallas as pl
from jax.experimental.pallas import tpu as pltpu
from jax.experimental.pallas import tpu_sc as plsc

D_MODEL = 64
ROWS_PER_BLOCK = 4


def _make_gather(b0: int, b1: int):
    mesh = plsc.VectorSubcoreMesh(core_axis_name="core", subcore_axis_name="subcore")
    blk = ROWS_PER_BLOCK * b1

    @jax.jit
    def gather(table, x):
        @pl.kernel(
            out_type=jax.ShapeDtypeStruct((b0 * b1, D_MODEL), table.dtype),
            mesh=mesh,
            compiler_params=pltpu.CompilerParams(use_tc_tiling_on_sc=False),
        )
        def k(table_hbm, idx_hbm, out_hbm):
            def body(idx_vmem, out_vmem):
                for j in range(ROWS_PER_BLOCK):
                    pltpu.sync_copy(
                        table_hbm.at[idx_vmem.at[j]],
                        out_vmem.at[pl.ds(j * b1, b1)],
                    )

            pltpu.emit_pipeline(
                body,
                grid=(b0 // ROWS_PER_BLOCK,),
                in_specs=[
                    pl.BlockSpec((ROWS_PER_BLOCK, b1), index_map=lambda i: (i, 0)),
                ],
                out_specs=[
                    pl.BlockSpec((blk, D_MODEL), index_map=lambda i: (i, 0)),
                ],
                core_axis_name=("core", "subcore"),
                dimension_semantics=(pltpu.PARALLEL,),
            )(idx_hbm, out_hbm)

        return k(table, x)

    return gather


def kernel(x, table):
    b0, b1 = x.shape
    out = _make_gather(b0, b1)(table, x.astype(jnp.int32))
    return out.reshape(b0, b1, D_MODEL)
